# Initial kernel scaffold; baseline (speedup 1.0000x reference)
import jax
import jax.numpy as jnp
from jax.experimental import pallas as pl


def kernel(x, embedding, ema_weight, ema_count):
    raise NotImplementedError("write your pallas kernel here")


# speedup vs baseline: 1.2236x; 1.2236x over previous
import functools

import jax
import jax.numpy as jnp
from jax.experimental import pallas as pl
from jax.experimental.pallas import tpu as pltpu

_N = 2
_M = 8192
_D = 32
_L = 32
_B = 128
_EMA_DECAY = 0.999
_EPS = 1e-05
_BETA = 0.25

_RB = 256
_NR = (_B * _L) // _RB


def _argmin_body(x_ref, et_ref, te_ref, tx_ref, idx_ref):
    xb = x_ref[0]
    et = et_ref[0]
    te = te_ref[0]
    tx = tx_ref[0]
    b = jax.lax.dot_general(xb, et, (((1,), (0,)), ((), ())),
                            precision=jax.lax.Precision.DEFAULT)
    dist = (te + tx.reshape(_RB, 1)) + (-2.0) * b
    mn = jnp.min(dist, axis=1, keepdims=True)
    iota = jax.lax.broadcasted_iota(jnp.int32, (_RB, _M), 1)
    idx = jnp.min(jnp.where(dist == mn, iota, _M), axis=1)
    idx_ref[0, 0, :] = idx


def _compute_indices(x_flat, embedding_t, te, tx, interpret=False):
    out = pl.pallas_call(
        _argmin_body,
        grid=(_N, _NR),
        in_specs=[
            pl.BlockSpec((1, _RB, _D), lambda n, r: (n, r, 0)),
            pl.BlockSpec((1, _D, _M), lambda n, r: (n, 0, 0)),
            pl.BlockSpec((1, 1, _M), lambda n, r: (n, 0, 0)),
            pl.BlockSpec((1, 1, _RB), lambda n, r: (n * _NR + r, 0, 0)),
        ],
        out_specs=pl.BlockSpec((1, 1, _RB), lambda n, r: (n * _NR + r, 0, 0)),
        out_shape=jax.ShapeDtypeStruct((_N * _NR, 1, _RB), jnp.int32),
        interpret=interpret,
    )(x_flat, embedding_t, te.reshape(_N, 1, _M), tx.reshape(_N * _NR, 1, _RB))
    return out.reshape(_N, _NR * _RB)


def kernel(x, embedding, ema_weight, ema_count):
    bs = x.shape[0]
    N, M, D = embedding.shape
    L = _L
    x4 = x.reshape(bs, N, D, L).reshape(bs, N * D, L, 1)
    B, C, H, W = x4.shape
    xr = jnp.transpose(x4.reshape(B, N, D, H, W), (1, 0, 3, 4, 2))
    x_flat = xr.reshape(N, B * H * W, D)

    te = jnp.sum(embedding ** 2, axis=2)
    tx = jnp.sum(x_flat ** 2, axis=2)
    embedding_t = jnp.swapaxes(embedding, 1, 2)
    indices = _compute_indices(x_flat, embedding_t, te, tx)

    encodings = jax.nn.one_hot(indices, M).astype(jnp.float32)
    indices_exp = jnp.broadcast_to(indices[:, :, None], (N, B * H * W, D))
    quantized = jnp.take_along_axis(embedding, indices_exp, axis=1)
    quantized = quantized.reshape(xr.shape)
    new_ema_count = _EMA_DECAY * ema_count + (1.0 - _EMA_DECAY) * jnp.sum(encodings, axis=1)
    n = jnp.sum(new_ema_count, axis=-1, keepdims=True)
    new_ema_count = (new_ema_count + _EPS) / (n + M * _EPS) * n
    encodings_t = jnp.swapaxes(encodings, 1, 2)
    dw = jax.lax.batch_matmul(encodings_t, x_flat)
    new_ema_weight = _EMA_DECAY * ema_weight + (1.0 - _EMA_DECAY) * dw
    new_embeddings = new_ema_weight / jnp.expand_dims(new_ema_count, axis=-1)
    e_latent_loss = jnp.mean((xr - quantized) ** 2)
    loss = _BETA * e_latent_loss
    avg_probs = jnp.mean(encodings, axis=1)
    perplexity = jnp.exp(-jnp.sum(avg_probs * jnp.log(avg_probs + 1e-10), axis=-1)).sum()
    q = jnp.transpose(quantized, (1, 0, 4, 2, 3)).reshape(B, N * D, H, W)
    inds = jnp.transpose(indices.reshape(N, B, H, W), (1, 0, 2, 3))
    z_q = q.reshape(bs, N * L * D)
    enc_q = q
    return (z_q, loss, perplexity, inds, enc_q, new_embeddings, new_ema_count, new_ema_weight)
